# baseline (speedup 1.0000x reference)
import jax
import jax.numpy as jnp
from jax import lax
from jax.experimental import pallas as pl
from jax.experimental.pallas import tpu as pltpu

CH = 10000
SUB = 4
SR = CH // SUB
K = 2


def _sub_copy(hbm_ref, bufs_ref, sems_ref, c, s):
    slot = lax.rem(c, K)
    return pltpu.make_async_copy(
        hbm_ref.at[pl.ds(c * CH + s * SR, SR), :],
        bufs_ref.at[slot, pl.ds(s * SR, SR)],
        sems_ref.at[slot, s],
    )


def _mlp_kernel(rx_hbm, tx_hbm,
                W_enc_ref, b_enc_ref,
                Wy1_ref, by1_ref, Wy2_ref, by2_ref,
                Wac1_ref, bac1_ref, Wac2_ref, bac2_ref,
                outy_ref, outac_ref,
                rbufs, tbufs, rsems, tsems,
                MyT_ref, cy_ref, MacT_ref, cac_ref):
    i = pl.program_id(0)
    nc = pl.num_programs(0)

    @pl.when(i == 0)
    def _prologue():
        for s in range(SUB):
            _sub_copy(rx_hbm, rbufs, rsems, 0, s).start()
            _sub_copy(tx_hbm, tbufs, tsems, 0, s).start()
        MyT_ref[...] = lax.dot_general(
            W_enc_ref[...], Wy1_ref[...], (((0,), (1,)), ((), ())),
            preferred_element_type=jnp.float32)
        cy_ref[...] = lax.dot_general(
            b_enc_ref[...], Wy1_ref[...], (((1,), (1,)), ((), ())),
            preferred_element_type=jnp.float32) + by1_ref[...]
        MacT_ref[...] = lax.dot_general(
            W_enc_ref[...], Wac1_ref[...], (((0,), (1,)), ((), ())),
            preferred_element_type=jnp.float32)
        cac_ref[...] = lax.dot_general(
            b_enc_ref[...], Wac1_ref[...], (((1,), (1,)), ((), ())),
            preferred_element_type=jnp.float32) + bac1_ref[...]

    @pl.when(i + 1 < nc)
    def _prefetch():
        for s in range(SUB):
            _sub_copy(rx_hbm, rbufs, rsems, i + 1, s).start()
            _sub_copy(tx_hbm, tbufs, tsems, i + 1, s).start()

    slot = lax.rem(i, K)
    for s in range(SUB):
        _sub_copy(rx_hbm, rbufs, rsems, i, s).wait()
        hy = jnp.maximum(
            jnp.dot(rbufs[slot, pl.ds(s * SR, SR)], MyT_ref[...],
                    preferred_element_type=jnp.float32) + cy_ref[...], 0.0)
        outy_ref[0, :, pl.ds(s * SR, SR)] = lax.dot_general(
            Wy2_ref[...], hy, (((1,), (1,)), ((), ())),
            preferred_element_type=jnp.float32) + by2_ref[...]

        _sub_copy(tx_hbm, tbufs, tsems, i, s).wait()
        hac = jnp.maximum(
            jnp.dot(tbufs[slot, pl.ds(s * SR, SR)], MacT_ref[...],
                    preferred_element_type=jnp.float32) + cac_ref[...], 0.0)
        outac_ref[0, :, pl.ds(s * SR, SR)] = lax.dot_general(
            Wac2_ref[...], hac, (((1,), (1,)), ((), ())),
            preferred_element_type=jnp.float32) + bac2_ref[...]


def kernel(mol_x, reaction_x, target_x, W_enc, b_enc, Wa1, ba1, Wa2, ba2,
           W_upd, b_upd, Wy1, by1, Wy2, by2, Wac1, bac1, Wac2, bac2):
    del mol_x, Wa1, ba1, Wa2, ba2, W_upd, b_upd
    n = reaction_x.shape[0]
    nc = n // CH

    b_enc2 = b_enc.reshape(1, -1)
    by1_2 = by1.reshape(1, -1)
    by2_2 = by2.reshape(1, 1)
    bac1_2 = bac1.reshape(1, -1)
    bac2_2 = bac2.reshape(1, 1)

    hbm_spec = pl.BlockSpec(memory_space=pltpu.MemorySpace.HBM)
    out_spec = pl.BlockSpec((1, 1, CH), lambda i: (i, 0, 0))

    def whole(shape):
        return pl.BlockSpec(shape, lambda i: tuple(0 for _ in shape))

    outy, outac = pl.pallas_call(
        _mlp_kernel,
        grid=(nc,),
        in_specs=[
            hbm_spec, hbm_spec,
            whole((128, 128)), whole((1, 128)),
            whole((64, 128)), whole((1, 64)), whole((1, 64)), whole((1, 1)),
            whole((64, 128)), whole((1, 64)), whole((1, 64)), whole((1, 1)),
        ],
        out_specs=[out_spec, out_spec],
        out_shape=[
            jax.ShapeDtypeStruct((nc, 1, CH), jnp.float32),
            jax.ShapeDtypeStruct((nc, 1, CH), jnp.float32),
        ],
        scratch_shapes=[
            pltpu.VMEM((K, CH, 128), jnp.float32),
            pltpu.VMEM((K, CH, 128), jnp.float32),
            pltpu.SemaphoreType.DMA((K, SUB)),
            pltpu.SemaphoreType.DMA((K, SUB)),
            pltpu.VMEM((128, 64), jnp.float32),
            pltpu.VMEM((1, 64), jnp.float32),
            pltpu.VMEM((128, 64), jnp.float32),
            pltpu.VMEM((1, 64), jnp.float32),
        ],
        compiler_params=pltpu.CompilerParams(
            dimension_semantics=("arbitrary",)),
    )(reaction_x, target_x,
      W_enc, b_enc2,
      Wy1, by1_2, Wy2, by2_2,
      Wac1, bac1_2, Wac2, bac2_2)

    return (outy.reshape(-1), outac.reshape(-1))


# speedup vs baseline: 1.8913x; 1.8913x over previous
import jax
import jax.numpy as jnp
from jax import lax
from jax.experimental import pallas as pl
from jax.experimental.pallas import tpu as pltpu

TILE = 10000


def _mlp_kernel(rx_ref, tx_ref,
                W_enc_ref, b_enc_ref,
                Wy1_ref, by1_ref, Wy2_ref, by2_ref,
                Wac1_ref, bac1_ref, Wac2_ref, bac2_ref,
                outy_ref, outac_ref):
    MyT = lax.dot_general(
        W_enc_ref[...], Wy1_ref[...], (((0,), (1,)), ((), ())),
        preferred_element_type=jnp.float32)
    cy = lax.dot_general(
        b_enc_ref[...], Wy1_ref[...], (((1,), (1,)), ((), ())),
        preferred_element_type=jnp.float32) + by1_ref[...]
    MacT = lax.dot_general(
        W_enc_ref[...], Wac1_ref[...], (((0,), (1,)), ((), ())),
        preferred_element_type=jnp.float32)
    cac = lax.dot_general(
        b_enc_ref[...], Wac1_ref[...], (((1,), (1,)), ((), ())),
        preferred_element_type=jnp.float32) + bac1_ref[...]

    hy = jnp.maximum(
        jnp.dot(rx_ref[...], MyT,
                preferred_element_type=jnp.float32) + cy, 0.0)
    outy_ref[0] = lax.dot_general(
        Wy2_ref[...], hy, (((1,), (1,)), ((), ())),
        preferred_element_type=jnp.float32) + by2_ref[...]

    hac = jnp.maximum(
        jnp.dot(tx_ref[...], MacT,
                preferred_element_type=jnp.float32) + cac, 0.0)
    outac_ref[0] = lax.dot_general(
        Wac2_ref[...], hac, (((1,), (1,)), ((), ())),
        preferred_element_type=jnp.float32) + bac2_ref[...]


def kernel(mol_x, reaction_x, target_x, W_enc, b_enc, Wa1, ba1, Wa2, ba2,
           W_upd, b_upd, Wy1, by1, Wy2, by2, Wac1, bac1, Wac2, bac2):
    del mol_x, Wa1, ba1, Wa2, ba2, W_upd, b_upd
    n = reaction_x.shape[0]
    nb = n // TILE

    b_enc2 = b_enc.reshape(1, -1)
    by1_2 = by1.reshape(1, -1)
    by2_2 = by2.reshape(1, 1)
    bac1_2 = bac1.reshape(1, -1)
    bac2_2 = bac2.reshape(1, 1)

    row_spec = pl.BlockSpec((TILE, 128), lambda i: (i, 0))
    out_spec = pl.BlockSpec((1, 1, TILE), lambda i: (i, 0, 0))

    def whole(shape):
        return pl.BlockSpec(shape, lambda i: tuple(0 for _ in shape))

    outy, outac = pl.pallas_call(
        _mlp_kernel,
        grid=(nb,),
        in_specs=[
            row_spec, row_spec,
            whole((128, 128)), whole((1, 128)),
            whole((64, 128)), whole((1, 64)), whole((1, 64)), whole((1, 1)),
            whole((64, 128)), whole((1, 64)), whole((1, 64)), whole((1, 1)),
        ],
        out_specs=[out_spec, out_spec],
        out_shape=[
            jax.ShapeDtypeStruct((nb, 1, TILE), jnp.float32),
            jax.ShapeDtypeStruct((nb, 1, TILE), jnp.float32),
        ],
        compiler_params=pltpu.CompilerParams(
            dimension_semantics=("parallel",)),
    )(reaction_x, target_x,
      W_enc, b_enc2,
      Wy1, by1_2, Wy2, by2_2,
      Wac1, bac1_2, Wac2, bac2_2)

    return (outy.reshape(-1), outac.reshape(-1))


# speedup vs baseline: 1.9313x; 1.0212x over previous
import jax
import jax.numpy as jnp
from jax import lax
from jax.experimental import pallas as pl
from jax.experimental.pallas import tpu as pltpu

TILE = 10000


def _mlp_kernel(rx_ref, tx_ref,
                W_enc_ref, b_enc_ref,
                Wy1_ref, by1_ref, Wy2_ref, by2_ref,
                Wac1_ref, bac1_ref, Wac2_ref, bac2_ref,
                outy_ref, outac_ref,
                MyT_ref, cy_ref, MacT_ref, cac_ref):
    i = pl.program_id(0)

    @pl.when(i == 0)
    def _fold_weights():
        MyT_ref[...] = lax.dot_general(
            W_enc_ref[...], Wy1_ref[...], (((0,), (1,)), ((), ())),
            preferred_element_type=jnp.float32)
        cy_ref[...] = lax.dot_general(
            b_enc_ref[...], Wy1_ref[...], (((1,), (1,)), ((), ())),
            preferred_element_type=jnp.float32) + by1_ref[...]
        MacT_ref[...] = lax.dot_general(
            W_enc_ref[...], Wac1_ref[...], (((0,), (1,)), ((), ())),
            preferred_element_type=jnp.float32)
        cac_ref[...] = lax.dot_general(
            b_enc_ref[...], Wac1_ref[...], (((1,), (1,)), ((), ())),
            preferred_element_type=jnp.float32) + bac1_ref[...]

    hy = jnp.maximum(
        jnp.dot(rx_ref[...], MyT_ref[...],
                preferred_element_type=jnp.float32) + cy_ref[...], 0.0)
    outy_ref[0] = lax.dot_general(
        Wy2_ref[...], hy, (((1,), (1,)), ((), ())),
        preferred_element_type=jnp.float32) + by2_ref[...]

    hac = jnp.maximum(
        jnp.dot(tx_ref[...], MacT_ref[...],
                preferred_element_type=jnp.float32) + cac_ref[...], 0.0)
    outac_ref[0] = lax.dot_general(
        Wac2_ref[...], hac, (((1,), (1,)), ((), ())),
        preferred_element_type=jnp.float32) + bac2_ref[...]


def kernel(mol_x, reaction_x, target_x, W_enc, b_enc, Wa1, ba1, Wa2, ba2,
           W_upd, b_upd, Wy1, by1, Wy2, by2, Wac1, bac1, Wac2, bac2):
    del mol_x, Wa1, ba1, Wa2, ba2, W_upd, b_upd
    n = reaction_x.shape[0]
    nb = n // TILE

    b_enc2 = b_enc.reshape(1, -1)
    by1_2 = by1.reshape(1, -1)
    by2_2 = by2.reshape(1, 1)
    bac1_2 = bac1.reshape(1, -1)
    bac2_2 = bac2.reshape(1, 1)

    row_spec = pl.BlockSpec((TILE, 128), lambda i: (i, 0))
    out_spec = pl.BlockSpec((1, 1, TILE), lambda i: (i, 0, 0))

    def whole(shape):
        return pl.BlockSpec(shape, lambda i: tuple(0 for _ in shape))

    outy, outac = pl.pallas_call(
        _mlp_kernel,
        grid=(nb,),
        in_specs=[
            row_spec, row_spec,
            whole((128, 128)), whole((1, 128)),
            whole((64, 128)), whole((1, 64)), whole((1, 64)), whole((1, 1)),
            whole((64, 128)), whole((1, 64)), whole((1, 64)), whole((1, 1)),
        ],
        out_specs=[out_spec, out_spec],
        out_shape=[
            jax.ShapeDtypeStruct((nb, 1, TILE), jnp.float32),
            jax.ShapeDtypeStruct((nb, 1, TILE), jnp.float32),
        ],
        scratch_shapes=[
            pltpu.VMEM((128, 64), jnp.float32),
            pltpu.VMEM((1, 64), jnp.float32),
            pltpu.VMEM((128, 64), jnp.float32),
            pltpu.VMEM((1, 64), jnp.float32),
        ],
        compiler_params=pltpu.CompilerParams(
            dimension_semantics=("arbitrary",)),
    )(reaction_x, target_x,
      W_enc, b_enc2,
      Wy1, by1_2, Wy2, by2_2,
      Wac1, bac1_2, Wac2, bac2_2)

    return (outy.reshape(-1), outac.reshape(-1))
